# baseline (speedup 1.0000x reference)
import functools

import jax
import jax.numpy as jnp
from jax import lax
from jax.experimental import pallas as pl
from jax.experimental.pallas import tpu as pltpu
from jax.experimental.pallas import tpu_sc as plsc

_N_EMBED = 1024
_DIM = 256
_B = 8
_HW = 1024
_TOKENS = _B * _HW
_BETA = 0.25
_T_TILE = 256


def _dist_argmin_body(z_ref, e_ref, idx_ref, md_ref):
    zb = z_ref[0]
    e = e_ref[...]
    s = lax.dot_general(e, zb, (((1,), (0,)), ((), ())),
                        preferred_element_type=jnp.float32)
    a = jnp.sum(zb * zb, axis=0, keepdims=True)
    b = jnp.sum(e * e, axis=1, keepdims=True)
    d = (a + b) - 2.0 * s
    m = jnp.min(d, axis=0)
    kk = lax.broadcasted_iota(jnp.int32, d.shape, 0)
    idx = jnp.min(jnp.where(d == m[None, :], kk, jnp.int32(2 ** 30)), axis=0)
    idx_ref[0, 0, :] = idx
    md_ref[0, 0, :] = m


def _dist_argmin(z3, emb):
    n_t = _HW // _T_TILE
    return pl.pallas_call(
        _dist_argmin_body,
        grid=(_B, n_t),
        in_specs=[
            pl.BlockSpec((1, _DIM, _T_TILE), lambda i, j: (i, 0, j)),
            pl.BlockSpec((_N_EMBED, _DIM), lambda i, j: (0, 0)),
        ],
        out_specs=[
            pl.BlockSpec((1, 1, _T_TILE), lambda i, j: (i, 0, j)),
            pl.BlockSpec((1, 1, _T_TILE), lambda i, j: (i, 0, j)),
        ],
        out_shape=[
            jax.ShapeDtypeStruct((_B, 1, _HW), jnp.int32),
            jax.ShapeDtypeStruct((_B, 1, _HW), jnp.float32),
        ],
    )(z3, emb)


@functools.lru_cache(maxsize=None)
def _make_sc_gather():
    mesh = plsc.VectorSubcoreMesh(core_axis_name="c", subcore_axis_name="s")

    @functools.partial(
        pl.kernel,
        mesh=mesh,
        out_type=jax.ShapeDtypeStruct((_B * _DIM, _HW), jnp.float32),
        compiler_params=pltpu.CompilerParams(needs_layout_passes=False),
        scratch_types=[
            pltpu.VMEM((8 * _HW,), jnp.float32),
            pltpu.VMEM((_TOKENS,), jnp.int32),
            pltpu.VMEM((8, _HW), jnp.float32),
            pltpu.VMEM((8, _HW), jnp.float32),
        ],
    )
    def sck(et_hbm, idx_hbm, z_hbm, out_hbm, et_v, idx_v, z_v, out_v):
        wid = lax.axis_index("s") * 2 + lax.axis_index("c")
        c0 = wid * 8
        pltpu.sync_copy(et_hbm.at[pl.ds(c0 * _HW, 8 * _HW)], et_v)
        pltpu.sync_copy(idx_hbm, idx_v)
        for b in range(_B):
            r0 = b * _DIM + c0
            pltpu.sync_copy(z_hbm.at[pl.ds(r0, 8)], z_v)

            def body(g, _):
                iv = idx_v[pl.ds(b * _HW + g * 16, 16)]
                for j in range(8):
                    gat = plsc.load_gather(et_v, [iv + jnp.int32(j * _HW)])
                    zvec = z_v[j, pl.ds(g * 16, 16)]
                    out_v[j, pl.ds(g * 16, 16)] = zvec + (gat - zvec)
                return _

            lax.fori_loop(0, _HW // 16, body, None)
            pltpu.sync_copy(out_v, out_hbm.at[pl.ds(r0, 8)])

    return sck


def kernel(z, embedding):
    z3 = z.reshape(_B, _DIM, _HW)
    idx3, md3 = _dist_argmin(z3, embedding)
    et = embedding.T.reshape(_DIM * _N_EMBED)
    out2d = _make_sc_gather()(et, idx3.reshape(_TOKENS),
                              z3.reshape(_B * _DIM, _HW))
    z_q_out = out2d.reshape(_B, _DIM, 32, 32)
    loss = _BETA * (jnp.sum(md3) / jnp.float32(_TOKENS * _DIM))
    return (z_q_out, loss)


# speedup vs baseline: 1.1857x; 1.1857x over previous
import functools

import jax
import jax.numpy as jnp
from jax import lax
from jax.experimental import pallas as pl
from jax.experimental.pallas import tpu as pltpu
from jax.experimental.pallas import tpu_sc as plsc

_N_EMBED = 1024
_DIM = 256
_B = 8
_HW = 1024
_TOKENS = _B * _HW
_BETA = 0.25
_T_TILE = 1024


def _dist_argmin_body(z_ref, e_ref, idx_ref, md_ref):
    zb = z_ref[0].reshape(_DIM, _HW)
    e = e_ref[...]
    s = lax.dot_general(e, zb, (((1,), (0,)), ((), ())),
                        preferred_element_type=jnp.float32)
    a = jnp.sum(zb * zb, axis=0, keepdims=True)
    b = jnp.sum(e * e, axis=1, keepdims=True)
    d = (a + b) - 2.0 * s
    m = jnp.min(d, axis=0)
    kk = lax.broadcasted_iota(jnp.int32, d.shape, 0)
    idx = jnp.min(jnp.where(d == m[None, :], kk, jnp.int32(2 ** 30)), axis=0)
    idx_ref[0, 0, :] = idx
    md_ref[0, 0, :] = m


def _dist_argmin(z, emb):
    return pl.pallas_call(
        _dist_argmin_body,
        grid=(_B,),
        in_specs=[
            pl.BlockSpec((1, _DIM, 32, 32), lambda i: (i, 0, 0, 0)),
            pl.BlockSpec((_N_EMBED, _DIM), lambda i: (0, 0)),
        ],
        out_specs=[
            pl.BlockSpec((1, 1, _T_TILE), lambda i: (i, 0, 0)),
            pl.BlockSpec((1, 1, _T_TILE), lambda i: (i, 0, 0)),
        ],
        out_shape=[
            jax.ShapeDtypeStruct((_B, 1, _HW), jnp.int32),
            jax.ShapeDtypeStruct((_B, 1, _HW), jnp.float32),
        ],
    )(z, emb)


@functools.lru_cache(maxsize=None)
def _make_sc_gather():
    mesh = plsc.VectorSubcoreMesh(core_axis_name="c", subcore_axis_name="s")

    @functools.partial(
        pl.kernel,
        mesh=mesh,
        out_type=jax.ShapeDtypeStruct((_B * _DIM, _HW), jnp.float32),
        compiler_params=pltpu.CompilerParams(needs_layout_passes=False),
        scratch_types=[
            pltpu.VMEM((8 * _HW,), jnp.float32),
            pltpu.VMEM((_TOKENS,), jnp.int32),
            pltpu.VMEM((8, _HW), jnp.float32),
            pltpu.VMEM((8, _HW), jnp.float32),
            pltpu.SemaphoreType.DMA,
            pltpu.SemaphoreType.DMA,
        ],
    )
    def sck(et_hbm, idx_hbm, out_hbm, et_v, idx_v, o0, o1, s0, s1):
        wid = lax.axis_index("s") * 2 + lax.axis_index("c")
        c0 = wid * 8
        pltpu.sync_copy(et_hbm.at[pl.ds(c0 * _HW, 8 * _HW)], et_v)
        pltpu.sync_copy(idx_hbm, idx_v)
        obufs, sems, pending = (o0, o1), (s0, s1), {}
        for b in range(_B):
            k = b % 2
            if k in pending:
                pending.pop(k).wait()
            ov = obufs[k]

            @plsc.parallel_loop(0, _HW // 16, unroll=4)
            def body(g):
                iv = idx_v[pl.ds(b * _HW + g * 16, 16)]
                for j in range(8):
                    gat = plsc.load_gather(et_v, [iv + jnp.int32(j * _HW)])
                    ov[j, pl.ds(g * 16, 16)] = gat

            pending[k] = pltpu.async_copy(
                ov, out_hbm.at[pl.ds(b * _DIM + c0, 8)], sems[k])
        for h in pending.values():
            h.wait()

    return sck


def _st_body(z_ref, zq_ref, out_ref):
    zb = z_ref[0]
    zq = zq_ref[...].reshape(_DIM, 32, 32)
    out_ref[0] = zb + (zq - zb)


def _st_add(z, zq2d):
    return pl.pallas_call(
        _st_body,
        grid=(_B,),
        in_specs=[
            pl.BlockSpec((1, _DIM, 32, 32), lambda i: (i, 0, 0, 0)),
            pl.BlockSpec((_DIM, _HW), lambda i: (i, 0)),
        ],
        out_specs=pl.BlockSpec((1, _DIM, 32, 32), lambda i: (i, 0, 0, 0)),
        out_shape=jax.ShapeDtypeStruct((_B, _DIM, 32, 32), jnp.float32),
    )(z, zq2d)


def kernel(z, embedding):
    idx3, md3 = _dist_argmin(z, embedding)
    et = embedding.T.reshape(_DIM * _N_EMBED)
    zq2d = _make_sc_gather()(et, idx3.reshape(_TOKENS))
    z_q_out = _st_add(z, zq2d)
    loss = _BETA * (jnp.sum(md3) / jnp.float32(_TOKENS * _DIM))
    return (z_q_out, loss)


# speedup vs baseline: 2.6727x; 2.2541x over previous
import functools

import jax
import jax.numpy as jnp
from jax import lax
from jax.experimental import pallas as pl
from jax.experimental.pallas import tpu as pltpu
from jax.experimental.pallas import tpu_sc as plsc

_N_EMBED = 1024
_DIM = 256
_B = 8
_HW = 1024
_TOKENS = _B * _HW
_BETA = 0.25
_T_TILE = 512
_N_TILES = _TOKENS // _T_TILE


def _dist_argmin_body(z_ref, e_ref, idx_ref, md_ref):
    zb = z_ref[...]
    e = e_ref[...]
    s = lax.dot_general(zb, e, (((1,), (1,)), ((), ())),
                        preferred_element_type=jnp.float32)
    a = jnp.sum(zb * zb, axis=1, keepdims=True)
    bb = jnp.sum(e * e, axis=1)
    d = (a + bb) - 2.0 * s
    m = jnp.min(d, axis=1)
    kk = lax.broadcasted_iota(jnp.int32, d.shape, 1)
    idx = jnp.min(jnp.where(d == m[:, None], kk, jnp.int32(2 ** 30)), axis=1)
    idx_ref[0, 0, :] = idx
    md_ref[0, 0, :] = m


def _dist_argmin(z2, emb):
    return pl.pallas_call(
        _dist_argmin_body,
        grid=(_N_TILES,),
        in_specs=[
            pl.BlockSpec((_T_TILE, _DIM), lambda i: (i, 0)),
            pl.BlockSpec((_N_EMBED, _DIM), lambda i: (0, 0)),
        ],
        out_specs=[
            pl.BlockSpec((1, 1, _T_TILE), lambda i: (i, 0, 0)),
            pl.BlockSpec((1, 1, _T_TILE), lambda i: (i, 0, 0)),
        ],
        out_shape=[
            jax.ShapeDtypeStruct((_N_TILES, 1, _T_TILE), jnp.int32),
            jax.ShapeDtypeStruct((_N_TILES, 1, _T_TILE), jnp.float32),
        ],
    )(z2, emb)


@functools.lru_cache(maxsize=None)
def _make_sc_gather():
    mesh = plsc.VectorSubcoreMesh(core_axis_name="c", subcore_axis_name="s")
    t_per_w = _TOKENS // 32

    @functools.partial(
        pl.kernel,
        mesh=mesh,
        out_type=jax.ShapeDtypeStruct((_TOKENS, _DIM), jnp.float32),
        scratch_types=[
            pltpu.VMEM((t_per_w,), jnp.int32),
            pltpu.VMEM((t_per_w, _DIM), jnp.float32),
            pltpu.SemaphoreType.DMA,
        ],
    )
    def sck(emb_hbm, idx_hbm, out_hbm, idx_v, rows_v, sem):
        wid = lax.axis_index("s") * 2 + lax.axis_index("c")
        base = wid * t_per_w
        pltpu.sync_copy(idx_hbm.at[pl.ds(base, t_per_w)], idx_v)
        pltpu.async_copy(emb_hbm.at[idx_v], rows_v, sem).wait()
        pltpu.sync_copy(rows_v, out_hbm.at[pl.ds(base, t_per_w)])

    return sck


def kernel(z, embedding):
    zp2 = jnp.transpose(z, (0, 2, 3, 1)).reshape(_TOKENS, _DIM)
    idx3, md3 = _dist_argmin(zp2, embedding)
    zq = _make_sc_gather()(embedding, idx3.reshape(_TOKENS))
    z_q_out = jnp.transpose(zq.reshape(_B, 32, 32, _DIM), (0, 3, 1, 2))
    loss = _BETA * (jnp.sum(md3) / jnp.float32(_TOKENS * _DIM))
    return (z_q_out, loss)


# speedup vs baseline: 2.6947x; 1.0082x over previous
import functools

import jax
import jax.numpy as jnp
from jax import lax
from jax.experimental import pallas as pl
from jax.experimental.pallas import tpu as pltpu
from jax.experimental.pallas import tpu_sc as plsc

_N_EMBED = 1024
_DIM = 256
_B = 8
_HW = 1024
_TOKENS = _B * _HW
_BETA = 0.25
_T_TILE = 512
_N_TILES = _TOKENS // _T_TILE


def _dist_argmin_body(z_ref, e2_ref, bb_ref, idx_ref, md_ref):
    zb = z_ref[...]
    e2 = e2_ref[...]
    bb = bb_ref[...]
    s2 = lax.dot_general(zb, e2, (((1,), (1,)), ((), ())),
                         preferred_element_type=jnp.float32)
    a = jnp.sum(zb * zb, axis=1, keepdims=True)
    d = (a + bb) - s2
    m = jnp.min(d, axis=1)
    kk = lax.broadcasted_iota(jnp.int32, d.shape, 1).astype(jnp.float32)
    idxf = jnp.min(jnp.where(d == m[:, None], kk, jnp.float32(2 ** 30)),
                   axis=1)
    idx_ref[0, 0, :] = idxf.astype(jnp.int32)
    md_ref[0, 0, :] = m


def _dist_argmin(z2, e2, bb):
    return pl.pallas_call(
        _dist_argmin_body,
        grid=(_N_TILES,),
        in_specs=[
            pl.BlockSpec((_T_TILE, _DIM), lambda i: (i, 0)),
            pl.BlockSpec((_N_EMBED, _DIM), lambda i: (0, 0)),
            pl.BlockSpec((1, _N_EMBED), lambda i: (0, 0)),
        ],
        out_specs=[
            pl.BlockSpec((1, 1, _T_TILE), lambda i: (i, 0, 0)),
            pl.BlockSpec((1, 1, _T_TILE), lambda i: (i, 0, 0)),
        ],
        out_shape=[
            jax.ShapeDtypeStruct((_N_TILES, 1, _T_TILE), jnp.int32),
            jax.ShapeDtypeStruct((_N_TILES, 1, _T_TILE), jnp.float32),
        ],
    )(z2, e2, bb)


@functools.lru_cache(maxsize=None)
def _make_sc_gather():
    mesh = plsc.VectorSubcoreMesh(core_axis_name="c", subcore_axis_name="s")
    t_per_w = _TOKENS // 32

    @functools.partial(
        pl.kernel,
        mesh=mesh,
        out_type=jax.ShapeDtypeStruct((_TOKENS, _DIM), jnp.float32),
        scratch_types=[
            pltpu.VMEM((t_per_w,), jnp.int32),
            pltpu.VMEM((t_per_w, _DIM), jnp.float32),
            pltpu.SemaphoreType.DMA,
        ],
    )
    def sck(emb_hbm, idx_hbm, out_hbm, idx_v, rows_v, sem):
        wid = lax.axis_index("s") * 2 + lax.axis_index("c")
        base = wid * t_per_w
        pltpu.sync_copy(idx_hbm.at[pl.ds(base, t_per_w)], idx_v)
        pltpu.async_copy(emb_hbm.at[idx_v], rows_v, sem).wait()
        pltpu.sync_copy(rows_v, out_hbm.at[pl.ds(base, t_per_w)])

    return sck


def kernel(z, embedding):
    zp2 = jnp.transpose(z, (0, 2, 3, 1)).reshape(_TOKENS, _DIM)
    e2 = embedding + embedding
    bb = jnp.sum(embedding * embedding, axis=1).reshape(1, _N_EMBED)
    idx3, md3 = _dist_argmin(zp2, e2, bb)
    zq = _make_sc_gather()(embedding, idx3.reshape(_TOKENS))
    z_q_out = jnp.transpose(zq.reshape(_B, 32, 32, _DIM), (0, 3, 1, 2))
    loss = _BETA * (jnp.sum(md3) / jnp.float32(_TOKENS * _DIM))
    return (z_q_out, loss)
